# baseline (speedup 1.0000x reference)
import functools

import jax
import jax.numpy as jnp
from jax import lax
from jax.experimental import pallas as pl
from jax.experimental.pallas import tpu as pltpu
from jax.experimental.pallas import tpu_sc as plsc


def kernel(input_ids, embedding_table, pos_emb_table):
    B, L = input_ids.shape
    V, D = embedding_table.shape
    N = B * L
    NW = 32
    n_per_w = N // NW
    CHUNK = 32
    n_chunks = n_per_w // CHUNK
    NBUF = 3
    n_groups = -(-n_chunks // NBUF)

    ids_flat = input_ids.reshape(N).astype(jnp.int32)
    mesh = plsc.VectorSubcoreMesh(core_axis_name="c", subcore_axis_name="s")

    @functools.partial(
        pl.kernel,
        mesh=mesh,
        out_type=jax.ShapeDtypeStruct((N, D), jnp.float32),
        scratch_types=[
            pltpu.VMEM((n_per_w,), jnp.int32),
            pltpu.VMEM((L, D), jnp.float32),
            pltpu.VMEM((CHUNK, D), jnp.float32),
            pltpu.VMEM((CHUNK, D), jnp.float32),
            pltpu.VMEM((CHUNK, D), jnp.float32),
            pltpu.SemaphoreType.DMA,
            pltpu.SemaphoreType.DMA,
            pltpu.SemaphoreType.DMA,
            pltpu.SemaphoreType.DMA,
            pltpu.SemaphoreType.DMA,
            pltpu.SemaphoreType.DMA,
        ],
    )
    def emb_kernel(ids_hbm, tab_hbm, pos_hbm, out_hbm,
                   idx_v, pos_v, buf0, buf1, buf2,
                   sg0, sg1, sg2, sw0, sw1, sw2):
        bufs = (buf0, buf1, buf2)
        semg = (sg0, sg1, sg2)
        semw = (sw0, sw1, sw2)
        wid = lax.axis_index("s") * 2 + lax.axis_index("c")
        base = wid * n_per_w
        pltpu.sync_copy(ids_hbm.at[pl.ds(base, n_per_w)], idx_v)
        pltpu.sync_copy(pos_hbm, pos_v)

        def start_gather(c, b):
            off = pl.multiple_of(c * CHUNK, CHUNK)
            pltpu.async_copy(tab_hbm.at[idx_v.at[pl.ds(off, CHUNK)]],
                             bufs[b], semg[b])

        start_gather(0, 0)
        start_gather(1, 1)

        def step(c, b):
            nb = (b + 2) % NBUF

            @pl.when((c >= 1) & (c <= n_chunks))
            def _():
                pltpu.make_async_copy(bufs[nb], out_hbm.at[pl.ds(0, CHUNK)],
                                      semw[nb]).wait()

            @pl.when(c + 2 < n_chunks)
            def _():
                start_gather(c + 2, nb)

            @pl.when(c < n_chunks)
            def _():
                buf = bufs[b]
                pltpu.make_async_copy(tab_hbm.at[idx_v.at[pl.ds(0, CHUNK)]],
                                      buf, semg[b]).wait()
                phase = (c % 2) * CHUNK

                def add_row(r, carry):
                    for j in range(D // 16):
                        s = pl.ds(j * 16, 16)
                        buf[r, s] = buf[r, s] + pos_v[phase + r, s]
                    return carry

                lax.fori_loop(0, CHUNK, add_row, 0)
                off = pl.multiple_of(c * CHUNK, CHUNK)
                pltpu.async_copy(buf, out_hbm.at[pl.ds(base + off, CHUNK)],
                                 semw[b])

        def group(g, carry):
            for b in range(NBUF):
                step(g * NBUF + b, b)
            return carry

        lax.fori_loop(0, n_groups, group, 0)

    out = emb_kernel(ids_flat, embedding_table, pos_emb_table)
    return out.reshape(B, L, D)


# speedup vs baseline: 1.0322x; 1.0322x over previous
import functools

import jax
import jax.numpy as jnp
from jax import lax
from jax.experimental import pallas as pl
from jax.experimental.pallas import tpu as pltpu
from jax.experimental.pallas import tpu_sc as plsc


def kernel(input_ids, embedding_table, pos_emb_table):
    B, L = input_ids.shape
    V, D = embedding_table.shape
    N = B * L
    NW = 32
    n_per_w = N // NW
    CHUNK = 32
    n_chunks = n_per_w // CHUNK
    NBUF = 3
    n_groups = (n_chunks - 4) // NBUF
    assert n_chunks == 4 + n_groups * NBUF

    ids_flat = input_ids.reshape(N).astype(jnp.int32)
    mesh = plsc.VectorSubcoreMesh(core_axis_name="c", subcore_axis_name="s")

    @functools.partial(
        pl.kernel,
        mesh=mesh,
        out_type=jax.ShapeDtypeStruct((N, D), jnp.float32),
        scratch_types=[
            pltpu.VMEM((n_per_w,), jnp.int32),
            pltpu.VMEM((L, D), jnp.float32),
            pltpu.VMEM((CHUNK, D), jnp.float32),
            pltpu.VMEM((CHUNK, D), jnp.float32),
            pltpu.VMEM((CHUNK, D), jnp.float32),
            pltpu.SemaphoreType.DMA,
            pltpu.SemaphoreType.DMA,
            pltpu.SemaphoreType.DMA,
            pltpu.SemaphoreType.DMA,
            pltpu.SemaphoreType.DMA,
            pltpu.SemaphoreType.DMA,
        ],
    )
    def emb_kernel(ids_hbm, tab_hbm, pos_hbm, out_hbm,
                   idx_v, pos_v, buf0, buf1, buf2,
                   sg0, sg1, sg2, sw0, sw1, sw2):
        bufs = (buf0, buf1, buf2)
        semg = (sg0, sg1, sg2)
        semw = (sw0, sw1, sw2)
        wid = lax.axis_index("s") * 2 + lax.axis_index("c")
        base = wid * n_per_w
        pltpu.sync_copy(ids_hbm.at[pl.ds(base, n_per_w)], idx_v)
        pltpu.sync_copy(pos_hbm, pos_v)

        def start_gather(c, b):
            off = pl.multiple_of(c * CHUNK, CHUNK)
            pltpu.async_copy(tab_hbm.at[idx_v.at[pl.ds(off, CHUNK)]],
                             bufs[b], semg[b])

        def wait_gather(b):
            pltpu.make_async_copy(tab_hbm.at[idx_v.at[pl.ds(0, CHUNK)]],
                                  bufs[b], semg[b]).wait()

        def wait_write(b):
            pltpu.make_async_copy(bufs[b], out_hbm.at[pl.ds(0, CHUNK)],
                                  semw[b]).wait()

        def process(c, b):
            buf = bufs[b]
            wait_gather(b)
            phase = (c % 2) * CHUNK

            def add_row(r, carry):
                for j in range(D // 16):
                    s = pl.ds(j * 16, 16)
                    buf[r, s] = buf[r, s] + pos_v[phase + r, s]
                return carry

            lax.fori_loop(0, CHUNK, add_row, 0)
            off = pl.multiple_of(c * CHUNK, CHUNK)
            pltpu.async_copy(buf, out_hbm.at[pl.ds(base + off, CHUNK)],
                             semw[b])

        start_gather(0, 0)
        start_gather(1, 1)
        process(0, 0)
        start_gather(2, 2)
        process(1, 1)
        wait_write(0)
        start_gather(3, 0)

        def group(g, carry):
            for k in range(NBUF):
                c = g * NBUF + 2 + k
                b = (2 + k) % NBUF
                nb = (b + 2) % NBUF
                wait_write(nb)
                start_gather(c + 2, nb)
                process(c, b)
            return carry

        lax.fori_loop(0, n_groups, group, 0)

        c0 = n_chunks - 2
        b0 = c0 % NBUF
        wait_write((b0 + 2) % NBUF)
        process(c0, b0)
        c1 = n_chunks - 1
        b1 = c1 % NBUF
        wait_write((b1 + 2) % NBUF)
        process(c1, b1)
        wait_write(b1)

    out = emb_kernel(ids_flat, embedding_table, pos_emb_table)
    return out.reshape(B, L, D)


# speedup vs baseline: 3.3673x; 3.2623x over previous
import functools

import jax
import jax.numpy as jnp
from jax import lax
from jax.experimental import pallas as pl
from jax.experimental.pallas import tpu as pltpu
from jax.experimental.pallas import tpu_sc as plsc


def kernel(input_ids, embedding_table, pos_emb_table):
    B, L = input_ids.shape
    V, D = embedding_table.shape
    N = B * L
    NW = 32
    LPW = L // NW
    n_per_w = LPW * B
    CHUNK = 64
    n_chunks = n_per_w // CHUNK
    cpl = B // CHUNK
    NS = D // 16

    ids_t = input_ids.T.reshape(N).astype(jnp.int32)
    mesh = plsc.VectorSubcoreMesh(core_axis_name="c", subcore_axis_name="s")

    @functools.partial(
        pl.kernel,
        mesh=mesh,
        out_type=jax.ShapeDtypeStruct((N, D), jnp.float32),
        scratch_types=[
            pltpu.VMEM((n_per_w,), jnp.int32),
            pltpu.VMEM((n_chunks, CHUNK), jnp.int32),
            pltpu.VMEM((LPW, D), jnp.float32),
            pltpu.VMEM((CHUNK, D), jnp.float32),
            pltpu.VMEM((CHUNK, D), jnp.float32),
            pltpu.SemaphoreType.DMA,
            pltpu.SemaphoreType.DMA,
            pltpu.SemaphoreType.DMA,
            pltpu.SemaphoreType.DMA,
        ],
    )
    def emb_kernel(ids_hbm, tab_hbm, pos_hbm, out_hbm,
                   idx_v, scat_idx, posbuf, buf0, buf1,
                   sg0, sg1, sw0, sw1):
        bufs = (buf0, buf1)
        semg = (sg0, sg1)
        semw = (sw0, sw1)
        wid = lax.axis_index("s") * 2 + lax.axis_index("c")
        l0 = wid * LPW
        base = wid * n_per_w
        pltpu.sync_copy(ids_hbm.at[pl.ds(base, n_per_w)], idx_v)
        pltpu.sync_copy(pos_hbm.at[pl.ds(l0, LPW)], posbuf)

        def build_scat(c, carry):
            lsel = c // cpl
            jj = c - lsel * cpl
            for kk in range(CHUNK // 16):
                v = (jj * CHUNK + kk * 16 + lax.iota(jnp.int32, 16)) * L
                scat_idx[c, pl.ds(kk * 16, 16)] = v + l0 + lsel
            return carry

        lax.fori_loop(0, n_chunks, build_scat, 0)

        def start_gather(c, b):
            off = pl.multiple_of(c * CHUNK, CHUNK)
            pltpu.async_copy(tab_hbm.at[idx_v.at[pl.ds(off, CHUNK)]],
                             bufs[b], semg[b])

        def wait_gather(b):
            pltpu.make_async_copy(tab_hbm.at[idx_v.at[pl.ds(0, CHUNK)]],
                                  bufs[b], semg[b]).wait()

        def start_write(c, b):
            pltpu.async_copy(bufs[b], out_hbm.at[scat_idx.at[c]], semw[b])

        def wait_write(b):
            pltpu.make_async_copy(bufs[b], out_hbm.at[scat_idx.at[0]],
                                  semw[b]).wait()

        def process(c, b):
            buf = bufs[b]
            wait_gather(b)
            lsel = c // cpl
            pos_regs = tuple(posbuf[lsel, pl.ds(i * 16, 16)]
                             for i in range(NS))

            def add_row(r, regs):
                for i in range(NS):
                    s = pl.ds(i * 16, 16)
                    buf[r, s] = buf[r, s] + regs[i]
                return regs

            lax.fori_loop(0, CHUNK, add_row, pos_regs)
            start_write(c, b)

        start_gather(0, 0)
        start_gather(1, 1)
        process(0, 0)

        def group(g, carry):
            for k in range(2):
                c = 2 * g + 1 + k
                b = (1 + k) % 2
                wait_write(1 - b)
                start_gather(c + 1, 1 - b)
                process(c, b)
            return carry

        lax.fori_loop(0, (n_chunks - 2) // 2, group, 0)

        cl = n_chunks - 1
        bl = cl % 2
        wait_write(1 - bl)
        process(cl, bl)
        wait_write(bl)

    out = emb_kernel(ids_t, embedding_table, pos_emb_table)
    return out.reshape(B, L, D)
